# baseline (speedup 1.0000x reference)
import jax
import jax.numpy as jnp
from jax import lax
from jax.experimental import pallas as pl
from jax.experimental.pallas import tpu as pltpu
from jax.experimental.pallas import tpu_sc as plsc

_EDIM = 32
_HDIM = 128
_PK = _EDIM // 2
_SPAD = 64
_GW = 2048
_BLK = 2048


def _sc_gather(table_i32, idx):
    n = idx.shape[0]
    mesh = plsc.VectorSubcoreMesh(core_axis_name="core",
                                  subcore_axis_name="subcore")
    idx2 = idx.reshape(1, n)
    v = table_i32.shape[0]

    @pl.kernel(out_type=jax.ShapeDtypeStruct((n, _PK), jnp.int32),
               mesh=mesh,
               scratch_types=[pltpu.VMEM_SHARED((v, _PK), jnp.int32)],
               compiler_params=pltpu.CompilerParams(use_tc_tiling_on_sc=False))
    def k(tab_hbm, i_hbm, o_hbm, tab_sp):
        @pl.when(lax.axis_index("subcore") == 0)
        def _():
            pltpu.sync_copy(tab_hbm, tab_sp)

        plsc.subcore_barrier()

        def body(i_vmem, o_vmem):
            pltpu.sync_copy(tab_sp.at[i_vmem.at[0]], o_vmem)

        pltpu.emit_pipeline(
            body,
            grid=(n // _GW,),
            in_specs=[pl.BlockSpec((1, _GW), index_map=lambda i: (0, i))],
            out_specs=[pl.BlockSpec((_GW, _PK), index_map=lambda i: (i, 0))],
            core_axis_name=("core", "subcore"),
            dimension_semantics=(pltpu.PARALLEL,),
        )(i_hbm, o_hbm)

    return k(table_i32, idx2)


def _mlp_body(x_ref, w1e_ref, w1o_ref, b1_ref, g_ref, be_ref, w2_ref, b2_ref,
              o_ref):
    w = x_ref[...]
    xe = lax.bitcast_convert_type(w << 16, jnp.float32)
    xo = lax.bitcast_convert_type(w & jnp.int32(-65536), jnp.float32)
    h = jnp.dot(xe.astype(jnp.bfloat16), w1e_ref[...],
                preferred_element_type=jnp.float32)
    h = h + jnp.dot(xo.astype(jnp.bfloat16), w1o_ref[...],
                    preferred_element_type=jnp.float32)
    h = h + b1_ref[...]
    mean = jnp.mean(h, axis=-1, keepdims=True)
    c = h - mean
    var = jnp.mean(c * c, axis=-1, keepdims=True)
    ln = c * lax.rsqrt(var + 1e-5) * g_ref[...] + be_ref[...]
    a = jnp.where(ln > 0, ln, jnp.exp(ln) - 1.0)
    out = jnp.dot(a.astype(jnp.bfloat16), w2_ref[...],
                  preferred_element_type=jnp.float32) + b2_ref[...]
    o_ref[...] = out


def _tc_mlp(x, w1e, w1o, b1, gamma, beta, w2, b2):
    b, d = x.shape
    grid = (b // _BLK,)
    full = lambda i: (0, 0)
    return pl.pallas_call(
        _mlp_body,
        grid=grid,
        in_specs=[
            pl.BlockSpec((_BLK, d), lambda i: (i, 0)),
            pl.BlockSpec(w1e.shape, full),
            pl.BlockSpec(w1o.shape, full),
            pl.BlockSpec((1, _HDIM), full),
            pl.BlockSpec((1, _HDIM), full),
            pl.BlockSpec((1, _HDIM), full),
            pl.BlockSpec(w2.shape, full),
            pl.BlockSpec((1, _HDIM), full),
        ],
        out_specs=pl.BlockSpec((_BLK, _HDIM), lambda i: (i, 0)),
        out_shape=jax.ShapeDtypeStruct((b, _HDIM), jnp.float32),
    )(x, w1e, w1o, b1, gamma, beta, w2, b2)


def kernel(inv_glyphs, emb, W1, b1, gamma, beta, W2, b2):
    bsz, slots = inv_glyphs.shape
    ig = inv_glyphs.astype(jnp.int32)
    pad = jnp.broadcast_to(ig[:, :1], (bsz, _SPAD - slots))
    idxp = jnp.concatenate([ig, pad], axis=1)
    idx = idxp.reshape(bsz // 8, 8, 8, 8).transpose(0, 2, 1, 3).reshape(-1)

    table_i32 = lax.bitcast_convert_type(
        emb.astype(jnp.bfloat16).reshape(-1, _PK, 2), jnp.int32)
    gathered = _sc_gather(table_i32, idx)
    x = (gathered.reshape(bsz // 8, 8, 8, 8, _PK)
         .transpose(0, 2, 1, 3, 4)
         .reshape(bsz, _SPAD * _PK))

    w1r = W1.astype(jnp.bfloat16).reshape(slots, _PK, 2, _HDIM)
    w1r = jnp.pad(w1r, ((0, _SPAD - slots), (0, 0), (0, 0), (0, 0)))
    w1e = w1r[:, :, 0, :].reshape(_SPAD * _PK, _HDIM)
    w1o = w1r[:, :, 1, :].reshape(_SPAD * _PK, _HDIM)

    return _tc_mlp(
        x,
        w1e,
        w1o,
        b1.reshape(1, _HDIM),
        gamma.reshape(1, _HDIM),
        beta.reshape(1, _HDIM),
        W2.astype(jnp.bfloat16),
        b2.reshape(1, _HDIM),
    )


# speedup vs baseline: 6.1278x; 6.1278x over previous
import jax
import jax.numpy as jnp
from jax import lax
from jax.experimental import pallas as pl
from jax.experimental.pallas import tpu as pltpu
from jax.experimental.pallas import tpu_sc as plsc

_EDIM = 32
_HDIM = 128
_PK = _EDIM // 2
_GW = 1760
_BLK = 2048


def _sc_gather(table_i32, idx):
    n = idx.shape[0]
    mesh = plsc.VectorSubcoreMesh(core_axis_name="core",
                                  subcore_axis_name="subcore")
    idx2 = idx.reshape(1, n)

    v = table_i32.shape[0]

    @pl.kernel(out_type=jax.ShapeDtypeStruct((n, _PK), jnp.int32),
               mesh=mesh,
               scratch_types=[pltpu.VMEM_SHARED((v, _PK), jnp.int32)],
               compiler_params=pltpu.CompilerParams(use_tc_tiling_on_sc=False))
    def k(tab_hbm, i_hbm, o_hbm, tab_sp):
        @pl.when(lax.axis_index("subcore") == 0)
        def _():
            pltpu.sync_copy(tab_hbm, tab_sp)

        plsc.subcore_barrier()

        def body(i_vmem, o_vmem):
            pltpu.sync_copy(tab_sp.at[i_vmem.at[0]], o_vmem)

        pltpu.emit_pipeline(
            body,
            grid=(n // _GW,),
            in_specs=[pl.BlockSpec((1, _GW), index_map=lambda i: (0, i))],
            out_specs=[pl.BlockSpec((_GW, _PK), index_map=lambda i: (i, 0))],
            core_axis_name=("core", "subcore"),
            dimension_semantics=(pltpu.PARALLEL,),
        )(i_hbm, o_hbm)

    return k(table_i32, idx2)


def _mlp_body(x_ref, w1e_ref, w1o_ref, b1_ref, g_ref, be_ref, w2_ref, b2_ref,
              o_ref):
    w = x_ref[...]
    xe = lax.bitcast_convert_type(w << 16, jnp.float32)
    xo = lax.bitcast_convert_type(w & jnp.int32(-65536), jnp.float32)
    h = jnp.dot(xe.astype(jnp.bfloat16), w1e_ref[...],
                preferred_element_type=jnp.float32)
    h = h + jnp.dot(xo.astype(jnp.bfloat16), w1o_ref[...],
                    preferred_element_type=jnp.float32)
    h = h + b1_ref[...]
    mean = jnp.mean(h, axis=-1, keepdims=True)
    c = h - mean
    var = jnp.mean(c * c, axis=-1, keepdims=True)
    ln = c * lax.rsqrt(var + 1e-5) * g_ref[...] + be_ref[...]
    a = jnp.where(ln > 0, ln, jnp.exp(ln) - 1.0)
    out = jnp.dot(a.astype(jnp.bfloat16), w2_ref[...],
                  preferred_element_type=jnp.float32) + b2_ref[...]
    o_ref[...] = out


def _tc_mlp(x, w1e, w1o, b1, gamma, beta, w2, b2):
    b, d = x.shape
    grid = (b // _BLK,)
    full = lambda i: (0, 0)
    return pl.pallas_call(
        _mlp_body,
        grid=grid,
        in_specs=[
            pl.BlockSpec((_BLK, d), lambda i: (i, 0)),
            pl.BlockSpec(w1e.shape, full),
            pl.BlockSpec(w1o.shape, full),
            pl.BlockSpec((1, _HDIM), full),
            pl.BlockSpec((1, _HDIM), full),
            pl.BlockSpec((1, _HDIM), full),
            pl.BlockSpec(w2.shape, full),
            pl.BlockSpec((1, _HDIM), full),
        ],
        out_specs=pl.BlockSpec((_BLK, _HDIM), lambda i: (i, 0)),
        out_shape=jax.ShapeDtypeStruct((b, _HDIM), jnp.float32),
    )(x, w1e, w1o, b1, gamma, beta, w2, b2)


_NCHUNKS = 4


def kernel(inv_glyphs, emb, W1, b1, gamma, beta, W2, b2):
    bsz, slots = inv_glyphs.shape
    idx = inv_glyphs.astype(jnp.int32).reshape(-1)
    table_i32 = lax.bitcast_convert_type(
        emb.astype(jnp.bfloat16).reshape(-1, _PK, 2), jnp.int32)
    w1p = W1.astype(jnp.bfloat16).reshape(slots * _PK, 2, _HDIM)
    w1e, w1o = w1p[:, 0, :], w1p[:, 1, :]
    b1r = b1.reshape(1, _HDIM)
    gr = gamma.reshape(1, _HDIM)
    ber = beta.reshape(1, _HDIM)
    w2b = W2.astype(jnp.bfloat16)
    b2r = b2.reshape(1, _HDIM)

    cb = bsz // _NCHUNKS
    cn = cb * slots
    outs = []
    for c in range(_NCHUNKS):
        gathered = _sc_gather(table_i32, idx[c * cn:(c + 1) * cn])
        x = gathered.reshape(cb, slots * _PK)
        outs.append(_tc_mlp(x, w1e, w1o, b1r, gr, ber, w2b, b2r))
    return jnp.concatenate(outs, axis=0)


# speedup vs baseline: 6.5518x; 1.0692x over previous
import jax
import jax.numpy as jnp
from jax import lax
from jax.experimental import pallas as pl
from jax.experimental.pallas import tpu as pltpu
from jax.experimental.pallas import tpu_sc as plsc

_EDIM = 32
_HDIM = 128
_PK = _EDIM // 2
_GW = 1760
_BLK = 2048


def _sc_gather(table_i32, idx):
    n = idx.shape[0]
    mesh = plsc.VectorSubcoreMesh(core_axis_name="core",
                                  subcore_axis_name="subcore")
    idx2 = idx.reshape(1, n)

    v = table_i32.shape[0]

    @pl.kernel(out_type=jax.ShapeDtypeStruct((n, _PK), jnp.int32),
               mesh=mesh,
               scratch_types=[pltpu.VMEM_SHARED((v, _PK), jnp.int32)],
               compiler_params=pltpu.CompilerParams(use_tc_tiling_on_sc=False))
    def k(tab_hbm, i_hbm, o_hbm, tab_sp):
        @pl.when(lax.axis_index("subcore") == 0)
        def _():
            pltpu.sync_copy(tab_hbm, tab_sp)

        plsc.subcore_barrier()

        def body(i_vmem, o_vmem):
            pltpu.sync_copy(tab_sp.at[i_vmem.at[0]], o_vmem)

        pltpu.emit_pipeline(
            body,
            grid=(n // _GW,),
            in_specs=[pl.BlockSpec((1, _GW), index_map=lambda i: (0, i))],
            out_specs=[pl.BlockSpec((_GW, _PK), index_map=lambda i: (i, 0))],
            core_axis_name=("core", "subcore"),
            dimension_semantics=(pltpu.PARALLEL,),
        )(i_hbm, o_hbm)

    return k(table_i32, idx2)


def _mlp_body(x_ref, w1e_ref, w1o_ref, b1_ref, g_ref, be_ref, w2_ref, b2_ref,
              o_ref):
    w = x_ref[...]
    xe = lax.bitcast_convert_type(w << 16, jnp.float32)
    xo = lax.bitcast_convert_type(w & jnp.int32(-65536), jnp.float32)
    h = jnp.dot(xe.astype(jnp.bfloat16), w1e_ref[...],
                preferred_element_type=jnp.float32)
    h = h + jnp.dot(xo.astype(jnp.bfloat16), w1o_ref[...],
                    preferred_element_type=jnp.float32)
    h = h + b1_ref[...]
    mean = jnp.mean(h, axis=-1, keepdims=True)
    c = h - mean
    var = jnp.mean(c * c, axis=-1, keepdims=True)
    ln = c * lax.rsqrt(var + 1e-5) * g_ref[...] + be_ref[...]
    a = jnp.where(ln > 0, ln, jnp.exp(ln) - 1.0)
    out = jnp.dot(a.astype(jnp.bfloat16), w2_ref[...],
                  preferred_element_type=jnp.float32) + b2_ref[...]
    o_ref[...] = out


def _tc_mlp(x, w1e, w1o, b1, gamma, beta, w2, b2):
    b, d = x.shape
    grid = (b // _BLK,)
    full = lambda i: (0, 0)
    return pl.pallas_call(
        _mlp_body,
        grid=grid,
        in_specs=[
            pl.BlockSpec((_BLK, d), lambda i: (i, 0)),
            pl.BlockSpec(w1e.shape, full),
            pl.BlockSpec(w1o.shape, full),
            pl.BlockSpec((1, _HDIM), full),
            pl.BlockSpec((1, _HDIM), full),
            pl.BlockSpec((1, _HDIM), full),
            pl.BlockSpec(w2.shape, full),
            pl.BlockSpec((1, _HDIM), full),
        ],
        out_specs=pl.BlockSpec((_BLK, _HDIM), lambda i: (i, 0)),
        out_shape=jax.ShapeDtypeStruct((b, _HDIM), jnp.float32),
    )(x, w1e, w1o, b1, gamma, beta, w2, b2)


_NCHUNKS = 2


def kernel(inv_glyphs, emb, W1, b1, gamma, beta, W2, b2):
    bsz, slots = inv_glyphs.shape
    idx = inv_glyphs.astype(jnp.int32).reshape(-1)
    table_i32 = lax.bitcast_convert_type(
        emb.astype(jnp.bfloat16).reshape(-1, _PK, 2), jnp.int32)
    w1p = W1.astype(jnp.bfloat16).reshape(slots * _PK, 2, _HDIM)
    w1e, w1o = w1p[:, 0, :], w1p[:, 1, :]
    b1r = b1.reshape(1, _HDIM)
    gr = gamma.reshape(1, _HDIM)
    ber = beta.reshape(1, _HDIM)
    w2b = W2.astype(jnp.bfloat16)
    b2r = b2.reshape(1, _HDIM)

    cb = bsz // _NCHUNKS
    cn = cb * slots
    outs = []
    for c in range(_NCHUNKS):
        gathered = _sc_gather(table_i32, idx[c * cn:(c + 1) * cn])
        x = gathered.reshape(cb, slots * _PK)
        outs.append(_tc_mlp(x, w1e, w1o, b1r, gr, ber, w2b, b2r))
    return jnp.concatenate(outs, axis=0)


# speedup vs baseline: 6.9966x; 1.0679x over previous
import jax
import jax.numpy as jnp
from jax import lax
from jax.experimental import pallas as pl
from jax.experimental.pallas import tpu as pltpu
from jax.experimental.pallas import tpu_sc as plsc

_EDIM = 32
_HDIM = 128
_PK = _EDIM // 2
_GW = 3520
_BLK = 4096


def _sc_gather(table_i32, idx):
    n = idx.shape[0]
    mesh = plsc.VectorSubcoreMesh(core_axis_name="core",
                                  subcore_axis_name="subcore")
    idx2 = idx.reshape(1, n)

    v = table_i32.shape[0]

    @pl.kernel(out_type=jax.ShapeDtypeStruct((n, _PK), jnp.int32),
               mesh=mesh,
               scratch_types=[pltpu.VMEM_SHARED((v, _PK), jnp.int32)],
               compiler_params=pltpu.CompilerParams(use_tc_tiling_on_sc=False))
    def k(tab_hbm, i_hbm, o_hbm, tab_sp):
        @pl.when(lax.axis_index("subcore") == 0)
        def _():
            pltpu.sync_copy(tab_hbm, tab_sp)

        plsc.subcore_barrier()

        def body(i_vmem, o_vmem):
            pltpu.sync_copy(tab_sp.at[i_vmem.at[0]], o_vmem)

        pltpu.emit_pipeline(
            body,
            grid=(n // _GW,),
            in_specs=[pl.BlockSpec((1, _GW), index_map=lambda i: (0, i))],
            out_specs=[pl.BlockSpec((_GW, _PK), index_map=lambda i: (i, 0))],
            core_axis_name=("core", "subcore"),
            dimension_semantics=(pltpu.PARALLEL,),
        )(i_hbm, o_hbm)

    return k(table_i32, idx2)


def _mlp_body(x_ref, w1e_ref, w1o_ref, b1_ref, g_ref, be_ref, w2_ref, b2_ref,
              o_ref):
    w = x_ref[...]
    xe = lax.bitcast_convert_type(w << 16, jnp.float32)
    xo = lax.bitcast_convert_type(w & jnp.int32(-65536), jnp.float32)
    h = jnp.dot(xe.astype(jnp.bfloat16), w1e_ref[...],
                preferred_element_type=jnp.float32)
    h = h + jnp.dot(xo.astype(jnp.bfloat16), w1o_ref[...],
                    preferred_element_type=jnp.float32)
    h = h + b1_ref[...]
    mean = jnp.mean(h, axis=-1, keepdims=True)
    c = h - mean
    var = jnp.mean(c * c, axis=-1, keepdims=True)
    ln = c * lax.rsqrt(var + 1e-5) * g_ref[...] + be_ref[...]
    a = jnp.where(ln > 0, ln, jnp.exp(ln) - 1.0)
    out = jnp.dot(a.astype(jnp.bfloat16), w2_ref[...],
                  preferred_element_type=jnp.float32) + b2_ref[...]
    o_ref[...] = out


def _tc_mlp(x, w1e, w1o, b1, gamma, beta, w2, b2):
    b, d = x.shape
    grid = (b // _BLK,)
    full = lambda i: (0, 0)
    return pl.pallas_call(
        _mlp_body,
        grid=grid,
        in_specs=[
            pl.BlockSpec((_BLK, d), lambda i: (i, 0)),
            pl.BlockSpec(w1e.shape, full),
            pl.BlockSpec(w1o.shape, full),
            pl.BlockSpec((1, _HDIM), full),
            pl.BlockSpec((1, _HDIM), full),
            pl.BlockSpec((1, _HDIM), full),
            pl.BlockSpec(w2.shape, full),
            pl.BlockSpec((1, _HDIM), full),
        ],
        out_specs=pl.BlockSpec((_BLK, _HDIM), lambda i: (i, 0)),
        out_shape=jax.ShapeDtypeStruct((b, _HDIM), jnp.float32),
    )(x, w1e, w1o, b1, gamma, beta, w2, b2)


def kernel(inv_glyphs, emb, W1, b1, gamma, beta, W2, b2):
    bsz, slots = inv_glyphs.shape
    idx = inv_glyphs.astype(jnp.int32).reshape(-1)
    table_i32 = lax.bitcast_convert_type(
        emb.astype(jnp.bfloat16).reshape(-1, _PK, 2), jnp.int32)
    gathered = _sc_gather(table_i32, idx)
    x = gathered.reshape(bsz, slots * _PK)
    w1p = W1.astype(jnp.bfloat16).reshape(slots * _PK, 2, _HDIM)
    return _tc_mlp(
        x,
        w1p[:, 0, :],
        w1p[:, 1, :],
        b1.reshape(1, _HDIM),
        gamma.reshape(1, _HDIM),
        beta.reshape(1, _HDIM),
        W2.astype(jnp.bfloat16),
        b2.reshape(1, _HDIM),
    )


# speedup vs baseline: 7.3241x; 1.0468x over previous
import jax
import jax.numpy as jnp
from jax import lax
from jax.experimental import pallas as pl
from jax.experimental.pallas import tpu as pltpu
from jax.experimental.pallas import tpu_sc as plsc

_EDIM = 32
_HDIM = 128
_PK = _EDIM // 2
_GW = 1792
_BLK = 4096


def _sc_gather(table_i32, idx):
    n = idx.shape[0]
    mesh = plsc.VectorSubcoreMesh(core_axis_name="core",
                                  subcore_axis_name="subcore")
    idx2 = idx.reshape(1, n)

    v = table_i32.shape[0]

    @pl.kernel(out_type=jax.ShapeDtypeStruct((n, _PK), jnp.int32),
               mesh=mesh,
               scratch_types=[pltpu.VMEM_SHARED((v, _PK), jnp.int32)],
               compiler_params=pltpu.CompilerParams(use_tc_tiling_on_sc=False))
    def k(tab_hbm, i_hbm, o_hbm, tab_sp):
        @pl.when(lax.axis_index("subcore") == 0)
        def _():
            pltpu.sync_copy(tab_hbm, tab_sp)

        plsc.subcore_barrier()

        def body(i_vmem, o_vmem):
            pltpu.sync_copy(tab_sp.at[i_vmem.at[0]], o_vmem)

        pltpu.emit_pipeline(
            body,
            grid=(n // _GW,),
            in_specs=[pl.BlockSpec((1, _GW), index_map=lambda i: (0, i))],
            out_specs=[pl.BlockSpec((_GW, _PK), index_map=lambda i: (i, 0))],
            core_axis_name=("core", "subcore"),
            dimension_semantics=(pltpu.PARALLEL,),
        )(i_hbm, o_hbm)

    return k(table_i32, idx2)


def _mlp_body(x_ref, w1e_ref, w1o_ref, b1_ref, g_ref, be_ref, w2_ref, b2_ref,
              o_ref):
    w = x_ref[...]
    xe = lax.bitcast_convert_type(w << 16, jnp.float32)
    xo = lax.bitcast_convert_type(w & jnp.int32(-65536), jnp.float32)
    h = jnp.dot(xe.astype(jnp.bfloat16), w1e_ref[...],
                preferred_element_type=jnp.float32)
    h = h + jnp.dot(xo.astype(jnp.bfloat16), w1o_ref[...],
                    preferred_element_type=jnp.float32)
    h = h + b1_ref[...]
    mean = jnp.mean(h, axis=-1, keepdims=True)
    c = h - mean
    var = jnp.mean(c * c, axis=-1, keepdims=True)
    ln = c * lax.rsqrt(var + 1e-5) * g_ref[...] + be_ref[...]
    a = jnp.where(ln > 0, ln, jnp.exp(ln) - 1.0)
    out = jnp.dot(a.astype(jnp.bfloat16), w2_ref[...],
                  preferred_element_type=jnp.float32) + b2_ref[...]
    o_ref[...] = out


def _tc_mlp(x, w1e, w1o, b1, gamma, beta, w2, b2):
    b, d = x.shape
    grid = (b // _BLK,)
    full = lambda i: (0, 0)
    return pl.pallas_call(
        _mlp_body,
        grid=grid,
        in_specs=[
            pl.BlockSpec((_BLK, d), lambda i: (i, 0)),
            pl.BlockSpec(w1e.shape, full),
            pl.BlockSpec(w1o.shape, full),
            pl.BlockSpec((1, _HDIM), full),
            pl.BlockSpec((1, _HDIM), full),
            pl.BlockSpec((1, _HDIM), full),
            pl.BlockSpec(w2.shape, full),
            pl.BlockSpec((1, _HDIM), full),
        ],
        out_specs=pl.BlockSpec((_BLK, _HDIM), lambda i: (i, 0)),
        out_shape=jax.ShapeDtypeStruct((b, _HDIM), jnp.float32),
    )(x, w1e, w1o, b1, gamma, beta, w2, b2)


def kernel(inv_glyphs, emb, W1, b1, gamma, beta, W2, b2):
    bsz, slots = inv_glyphs.shape
    ig = inv_glyphs.astype(jnp.int32)
    idx = jnp.concatenate([ig, ig[:, :1]], axis=1).reshape(-1)
    spad = slots + 1
    table_i32 = lax.bitcast_convert_type(
        emb.astype(jnp.bfloat16).reshape(-1, _PK, 2), jnp.int32)
    gathered = _sc_gather(table_i32, idx)
    x = gathered.reshape(bsz, spad * _PK)
    w1p = W1.astype(jnp.bfloat16).reshape(slots, _PK, 2, _HDIM)
    w1p = jnp.pad(w1p, ((0, 1), (0, 0), (0, 0), (0, 0)))
    return _tc_mlp(
        x,
        w1p[:, :, 0, :].reshape(spad * _PK, _HDIM),
        w1p[:, :, 1, :].reshape(spad * _PK, _HDIM),
        b1.reshape(1, _HDIM),
        gamma.reshape(1, _HDIM),
        beta.reshape(1, _HDIM),
        W2.astype(jnp.bfloat16),
        b2.reshape(1, _HDIM),
    )
